# baseline (speedup 1.0000x reference)
import jax
import jax.numpy as jnp
from jax.experimental import pallas as pl
from jax.experimental.pallas import tpu as pltpu

_TM = 2048
_NBUF = 3


def _streamed_linear_kernel(x_hbm, w_hbm, b_hbm, o_hbm,
                            xbuf, obuf, wbuf, bbuf,
                            xsem, osem, wsem, bsem):
    nch = x_hbm.shape[0] // _TM

    def xcp(i, slot):
        return pltpu.make_async_copy(
            x_hbm.at[pl.ds(i * _TM, _TM), :], xbuf.at[slot], xsem.at[slot])

    def ocp(i, slot):
        return pltpu.make_async_copy(
            obuf.at[slot], o_hbm.at[pl.ds(i * _TM, _TM), :], osem.at[slot])

    wcp = pltpu.make_async_copy(w_hbm, wbuf, wsem)
    bcp = pltpu.make_async_copy(b_hbm, bbuf, bsem)
    wcp.start()
    bcp.start()
    xcp(0, 0).start()
    xcp(1, 1).start()
    wcp.wait()
    bcp.wait()

    for i in range(nch):
        slot = i % _NBUF
        if i + 2 < nch:
            xcp(i + 2, (i + 2) % _NBUF).start()
        xcp(i, slot).wait()
        if i >= _NBUF:
            ocp(i - _NBUF, slot).wait()
        obuf[slot] = (
            jnp.dot(xbuf[slot], wbuf[...], preferred_element_type=jnp.float32)
            + bbuf[...]
        ).astype(obuf.dtype)
        ocp(i, slot).start()

    for i in range(max(nch - _NBUF, 0), nch):
        ocp(i, i % _NBUF).wait()


def kernel(x, w, b):
    M, K = x.shape
    _, N = w.shape
    out_dtype = x.dtype
    b2 = b.astype(jnp.float32).reshape(1, N)

    Mp = -(-M // _TM) * _TM
    if Mp != M:
        x = jnp.pad(x, ((0, Mp - M), (0, 0)))

    cost = pl.CostEstimate(
        flops=2 * Mp * K * N,
        bytes_accessed=Mp * K * 4 + K * N * 4 + Mp * N * 4 + N * 4,
        transcendentals=0,
    )
    out = pl.pallas_call(
        _streamed_linear_kernel,
        out_shape=jax.ShapeDtypeStruct((Mp, N), out_dtype),
        in_specs=[
            pl.BlockSpec(memory_space=pltpu.MemorySpace.HBM),
            pl.BlockSpec(memory_space=pltpu.MemorySpace.HBM),
            pl.BlockSpec(memory_space=pltpu.MemorySpace.HBM),
        ],
        out_specs=pl.BlockSpec(memory_space=pltpu.MemorySpace.HBM),
        scratch_shapes=[
            pltpu.VMEM((_NBUF, _TM, K), jnp.float32),
            pltpu.VMEM((_NBUF, _TM, N), jnp.float32),
            pltpu.VMEM((K, N), jnp.float32),
            pltpu.VMEM((1, N), jnp.float32),
            pltpu.SemaphoreType.DMA((_NBUF,)),
            pltpu.SemaphoreType.DMA((_NBUF,)),
            pltpu.SemaphoreType.DMA,
            pltpu.SemaphoreType.DMA,
        ],
        compiler_params=pltpu.CompilerParams(
            vmem_limit_bytes=61440000,
        ),
        cost_estimate=cost,
    )(x, w, b2)
    if Mp != M:
        out = out[:M]
    return out


# speedup vs baseline: 1.0292x; 1.0292x over previous
import jax
import jax.numpy as jnp
from jax.experimental import pallas as pl
from jax.experimental.pallas import tpu as pltpu

_TM = 1024
_NBUF = 4


def _streamed_linear_kernel(x_hbm, w_hbm, b_hbm, o_hbm,
                            xbuf, obuf, wbuf, bbuf,
                            xsem, osem, wsem, bsem):
    nch = x_hbm.shape[0] // _TM

    def xcp(i, slot):
        return pltpu.make_async_copy(
            x_hbm.at[pl.ds(i * _TM, _TM), :], xbuf.at[slot], xsem.at[slot])

    def ocp(i, slot):
        return pltpu.make_async_copy(
            obuf.at[slot], o_hbm.at[pl.ds(i * _TM, _TM), :], osem.at[slot])

    wcp = pltpu.make_async_copy(w_hbm, wbuf, wsem)
    bcp = pltpu.make_async_copy(b_hbm, bbuf, bsem)
    wcp.start()
    bcp.start()
    xcp(0, 0).start()
    xcp(1, 1).start()
    xcp(2, 2).start()
    wcp.wait()
    bcp.wait()

    for i in range(nch):
        slot = i % _NBUF
        if i + 3 < nch:
            xcp(i + 3, (i + 3) % _NBUF).start()
        xcp(i, slot).wait()
        if i >= _NBUF:
            ocp(i - _NBUF, slot).wait()
        obuf[slot] = (
            jnp.dot(xbuf[slot], wbuf[...], preferred_element_type=jnp.float32)
            + bbuf[...]
        ).astype(obuf.dtype)
        ocp(i, slot).start()

    for i in range(max(nch - _NBUF, 0), nch):
        ocp(i, i % _NBUF).wait()


def kernel(x, w, b):
    M, K = x.shape
    _, N = w.shape
    out_dtype = x.dtype
    b2 = b.astype(jnp.float32).reshape(1, N)

    Mp = -(-M // _TM) * _TM
    if Mp != M:
        x = jnp.pad(x, ((0, Mp - M), (0, 0)))

    cost = pl.CostEstimate(
        flops=2 * Mp * K * N,
        bytes_accessed=Mp * K * 4 + K * N * 4 + Mp * N * 4 + N * 4,
        transcendentals=0,
    )
    out = pl.pallas_call(
        _streamed_linear_kernel,
        out_shape=jax.ShapeDtypeStruct((Mp, N), out_dtype),
        in_specs=[
            pl.BlockSpec(memory_space=pltpu.MemorySpace.HBM),
            pl.BlockSpec(memory_space=pltpu.MemorySpace.HBM),
            pl.BlockSpec(memory_space=pltpu.MemorySpace.HBM),
        ],
        out_specs=pl.BlockSpec(memory_space=pltpu.MemorySpace.HBM),
        scratch_shapes=[
            pltpu.VMEM((_NBUF, _TM, K), jnp.float32),
            pltpu.VMEM((_NBUF, _TM, N), jnp.float32),
            pltpu.VMEM((K, N), jnp.float32),
            pltpu.VMEM((1, N), jnp.float32),
            pltpu.SemaphoreType.DMA((_NBUF,)),
            pltpu.SemaphoreType.DMA((_NBUF,)),
            pltpu.SemaphoreType.DMA,
            pltpu.SemaphoreType.DMA,
        ],
        compiler_params=pltpu.CompilerParams(
            vmem_limit_bytes=61440000,
        ),
        cost_estimate=cost,
    )(x, w, b2)
    if Mp != M:
        out = out[:M]
    return out


# speedup vs baseline: 1.0921x; 1.0611x over previous
import jax
import jax.numpy as jnp
from jax.experimental import pallas as pl
from jax.experimental.pallas import tpu as pltpu

_TM = 1024
_NBUF = 3


def _chunks(m):
    spans = []
    off = 0
    while off < m:
        size = min(_TM, m - off)
        spans.append((off, size))
        off += size
    if len(spans) > 1 and spans[-1][1] == _TM:
        off, _ = spans[-1]
        half = _TM // 2
        spans[-1] = (off, half)
        spans.append((off + half, half))
    return spans


def _streamed_linear_kernel(x_hbm, w_hbm, b_hbm, o_hbm,
                            xbuf, obuf, wbuf, bbuf,
                            xsem, osem, wsem, bsem):
    spans = _chunks(x_hbm.shape[0])
    nch = len(spans)

    def xcp(i, slot):
        off, size = spans[i]
        return pltpu.make_async_copy(
            x_hbm.at[pl.ds(off, size), :],
            xbuf.at[slot, pl.ds(0, size), :],
            xsem.at[slot])

    def ocp(i, slot):
        off, size = spans[i]
        return pltpu.make_async_copy(
            obuf.at[slot, pl.ds(0, size), :],
            o_hbm.at[pl.ds(off, size), :],
            osem.at[slot])

    wcp = pltpu.make_async_copy(w_hbm, wbuf, wsem)
    bcp = pltpu.make_async_copy(b_hbm, bbuf, bsem)
    wcp.start()
    bcp.start()
    xcp(0, 0).start()
    xcp(1, 1).start()
    wcp.wait()
    bcp.wait()

    for i in range(nch):
        slot = i % _NBUF
        if i + 2 < nch:
            xcp(i + 2, (i + 2) % _NBUF).start()
        xcp(i, slot).wait()
        if i >= _NBUF:
            ocp(i - _NBUF, slot).wait()
        size = spans[i][1]
        obuf[slot, pl.ds(0, size), :] = (
            jnp.dot(xbuf[slot, pl.ds(0, size), :], wbuf[...],
                    preferred_element_type=jnp.float32)
            + bbuf[...]
        ).astype(obuf.dtype)
        ocp(i, slot).start()

    for i in range(max(nch - _NBUF, 0), nch):
        ocp(i, i % _NBUF).wait()


def kernel(x, w, b):
    M, K = x.shape
    _, N = w.shape
    out_dtype = x.dtype
    b2 = b.astype(jnp.float32).reshape(1, N)

    Mp = -(-M // _TM) * _TM
    if Mp != M:
        x = jnp.pad(x, ((0, Mp - M), (0, 0)))

    cost = pl.CostEstimate(
        flops=2 * Mp * K * N,
        bytes_accessed=Mp * K * 4 + K * N * 4 + Mp * N * 4 + N * 4,
        transcendentals=0,
    )
    out = pl.pallas_call(
        _streamed_linear_kernel,
        out_shape=jax.ShapeDtypeStruct((Mp, N), out_dtype),
        in_specs=[
            pl.BlockSpec(memory_space=pltpu.MemorySpace.HBM),
            pl.BlockSpec(memory_space=pltpu.MemorySpace.HBM),
            pl.BlockSpec(memory_space=pltpu.MemorySpace.HBM),
        ],
        out_specs=pl.BlockSpec(memory_space=pltpu.MemorySpace.HBM),
        scratch_shapes=[
            pltpu.VMEM((_NBUF, _TM, K), jnp.float32),
            pltpu.VMEM((_NBUF, _TM, N), jnp.float32),
            pltpu.VMEM((K, N), jnp.float32),
            pltpu.VMEM((1, N), jnp.float32),
            pltpu.SemaphoreType.DMA((_NBUF,)),
            pltpu.SemaphoreType.DMA((_NBUF,)),
            pltpu.SemaphoreType.DMA,
            pltpu.SemaphoreType.DMA,
        ],
        compiler_params=pltpu.CompilerParams(
            vmem_limit_bytes=61440000,
        ),
        cost_estimate=cost,
    )(x, w, b2)
    if Mp != M:
        out = out[:M]
    return out


# speedup vs baseline: 1.1052x; 1.0120x over previous
import jax
import jax.numpy as jnp
from jax.experimental import pallas as pl
from jax.experimental.pallas import tpu as pltpu

_TM = 1024
_NBUF = 3


def _chunks(m):
    spans = []
    off = 0
    while off < m:
        size = min(_TM, m - off)
        spans.append((off, size))
        off += size
    half = _TM // 2
    if len(spans) > 1 and spans[-1][1] == _TM:
        off, _ = spans[-1]
        spans[-1] = (off, half)
        spans.append((off + half, half))
    if len(spans) > 1 and spans[0][1] == _TM:
        spans[0] = (0, half)
        spans.insert(1, (half, half))
    return spans


def _streamed_linear_kernel(x_hbm, w_hbm, b_hbm, o_hbm,
                            xbuf, obuf, wbuf, bbuf,
                            xsem, osem, wsem, bsem):
    spans = _chunks(x_hbm.shape[0])
    nch = len(spans)

    def xcp(i, slot):
        off, size = spans[i]
        return pltpu.make_async_copy(
            x_hbm.at[pl.ds(off, size), :],
            xbuf.at[slot, pl.ds(0, size), :],
            xsem.at[slot])

    def ocp(i, slot):
        off, size = spans[i]
        return pltpu.make_async_copy(
            obuf.at[slot, pl.ds(0, size), :],
            o_hbm.at[pl.ds(off, size), :],
            osem.at[slot])

    wcp = pltpu.make_async_copy(w_hbm, wbuf, wsem)
    bcp = pltpu.make_async_copy(b_hbm, bbuf, bsem)
    wcp.start()
    bcp.start()
    xcp(0, 0).start()
    xcp(1, 1).start()
    wcp.wait()
    bcp.wait()

    for i in range(nch):
        slot = i % _NBUF
        if i + 2 < nch:
            xcp(i + 2, (i + 2) % _NBUF).start()
        xcp(i, slot).wait()
        if i >= _NBUF:
            ocp(i - _NBUF, slot).wait()
        size = spans[i][1]
        obuf[slot, pl.ds(0, size), :] = (
            jnp.dot(xbuf[slot, pl.ds(0, size), :], wbuf[...],
                    preferred_element_type=jnp.float32)
            + bbuf[...]
        ).astype(obuf.dtype)
        ocp(i, slot).start()

    for i in range(max(nch - _NBUF, 0), nch):
        ocp(i, i % _NBUF).wait()


def kernel(x, w, b):
    M, K = x.shape
    _, N = w.shape
    out_dtype = x.dtype
    b2 = b.astype(jnp.float32).reshape(1, N)

    Mp = -(-M // _TM) * _TM
    if Mp != M:
        x = jnp.pad(x, ((0, Mp - M), (0, 0)))

    cost = pl.CostEstimate(
        flops=2 * Mp * K * N,
        bytes_accessed=Mp * K * 4 + K * N * 4 + Mp * N * 4 + N * 4,
        transcendentals=0,
    )
    out = pl.pallas_call(
        _streamed_linear_kernel,
        out_shape=jax.ShapeDtypeStruct((Mp, N), out_dtype),
        in_specs=[
            pl.BlockSpec(memory_space=pltpu.MemorySpace.HBM),
            pl.BlockSpec(memory_space=pltpu.MemorySpace.HBM),
            pl.BlockSpec(memory_space=pltpu.MemorySpace.HBM),
        ],
        out_specs=pl.BlockSpec(memory_space=pltpu.MemorySpace.HBM),
        scratch_shapes=[
            pltpu.VMEM((_NBUF, _TM, K), jnp.float32),
            pltpu.VMEM((_NBUF, _TM, N), jnp.float32),
            pltpu.VMEM((K, N), jnp.float32),
            pltpu.VMEM((1, N), jnp.float32),
            pltpu.SemaphoreType.DMA((_NBUF,)),
            pltpu.SemaphoreType.DMA((_NBUF,)),
            pltpu.SemaphoreType.DMA,
            pltpu.SemaphoreType.DMA,
        ],
        compiler_params=pltpu.CompilerParams(
            vmem_limit_bytes=61440000,
        ),
        cost_estimate=cost,
    )(x, w, b2)
    if Mp != M:
        out = out[:M]
    return out
